# baseline (speedup 1.0000x reference)
import functools
import math

import jax
import jax.numpy as jnp
from jax import lax
from jax.experimental import pallas as pl
from jax.experimental.pallas import tpu as pltpu
from jax.experimental.pallas import tpu_sc as plsc

B_DIM = 16384
T_DIM = 50
D_MODEL = 64
DW = 128
NUM_TOKENS = B_DIM * T_DIM
VOCAB = 1_000_000
SCALE = math.sqrt(D_MODEL)

NC, NS = 2, 16
NW = NC * NS
TOK_PER_W = NUM_TOKENS // NW
SUB = 128
NBUF = 4
NCHUNK = TOK_PER_W // SUB
IDXROWS_W = NCHUNK

PREP_ROWS = 8000


def _prep_body(tbl_ref, out_ref):
    r0 = pl.program_id(0) * PREP_ROWS
    x = tbl_ref[...] * SCALE
    row = r0 + lax.broadcasted_iota(jnp.int32, x.shape, 0)
    x = jnp.where(row == 0, 0.0, x)
    out_ref[...] = jnp.concatenate(
        [x, jnp.zeros_like(x)], axis=1)


_prep_kernel = pl.pallas_call(
    _prep_body,
    grid=(VOCAB // PREP_ROWS,),
    in_specs=[pl.BlockSpec((PREP_ROWS, D_MODEL), lambda i: (i, 0))],
    out_specs=pl.BlockSpec((PREP_ROWS, DW), lambda i: (i, 0)),
    out_shape=jax.ShapeDtypeStruct((VOCAB, DW), jnp.float32),
)


def _gather_body(table_hbm, idx_hbm, out_hbm,
                 idx_all, rows0, rows1, rows2, rows3,
                 gsem0, gsem1, gsem2, gsem3, ssem0, ssem1, ssem2, ssem3):
    rows = (rows0, rows1, rows2, rows3)
    gsem = (gsem0, gsem1, gsem2, gsem3)
    ssem = (ssem0, ssem1, ssem2, ssem3)

    wid = lax.axis_index("s") * NC + lax.axis_index("c")
    base = wid * TOK_PER_W
    idx_row0 = wid * IDXROWS_W

    pltpu.sync_copy(idx_hbm.at[pl.ds(idx_row0, IDXROWS_W)], idx_all)

    def fire_gather(chunk, buf):
        pltpu.async_copy(
            table_hbm.at[idx_all.at[chunk]], rows[buf], gsem[buf])

    def wait_gather(buf):
        pltpu.make_async_copy(
            table_hbm.at[idx_all.at[0]], rows[buf], gsem[buf]).wait()

    def fire_store(chunk, buf):
        pltpu.async_copy(
            rows[buf], out_hbm.at[pl.ds(base + chunk * SUB, SUB)], ssem[buf])

    def wait_store(buf):
        pltpu.make_async_copy(
            rows[buf], out_hbm.at[pl.ds(base, SUB)], ssem[buf]).wait()

    for k in range(NBUF - 1):
        fire_gather(k, k)

    @pl.loop(0, NCHUNK, step=NBUF)
    def _pipeline(c):
        for b in range(NBUF):
            cc = c + b
            nb = (b + NBUF - 1) % NBUF

            @pl.when(cc >= 1)
            def _():
                wait_store(nb)

            nxt = jnp.minimum(cc + NBUF - 1, NCHUNK - 1)
            fire_gather(nxt, nb)

            wait_gather(b)
            fire_store(cc, b)

    wait_store((NCHUNK - 1) % NBUF)
    for k in range(NBUF - 1):
        wait_gather((NCHUNK + k) % NBUF)


_gather_kernel = functools.partial(
    pl.kernel,
    mesh=plsc.VectorSubcoreMesh(core_axis_name="c", subcore_axis_name="s"),
    out_type=jax.ShapeDtypeStruct((NUM_TOKENS, DW), jnp.float32),
    compiler_params=pltpu.CompilerParams(use_tc_tiling_on_sc=True),
    scratch_types=[
        pltpu.VMEM((IDXROWS_W, SUB), jnp.int32),
        pltpu.VMEM((SUB, DW), jnp.float32),
        pltpu.VMEM((SUB, DW), jnp.float32),
        pltpu.VMEM((SUB, DW), jnp.float32),
        pltpu.VMEM((SUB, DW), jnp.float32),
        pltpu.SemaphoreType.DMA,
        pltpu.SemaphoreType.DMA,
        pltpu.SemaphoreType.DMA,
        pltpu.SemaphoreType.DMA,
        pltpu.SemaphoreType.DMA,
        pltpu.SemaphoreType.DMA,
        pltpu.SemaphoreType.DMA,
        pltpu.SemaphoreType.DMA,
    ],
)(_gather_body)


def kernel(inputs, table):
    table128 = _prep_kernel(table)
    idx2d = inputs.astype(jnp.int32).reshape(NUM_TOKENS // SUB, SUB)
    raw = _gather_kernel(table128, idx2d)
    return raw[:, :D_MODEL].reshape(B_DIM, T_DIM, D_MODEL)
